# Initial kernel scaffold; baseline (speedup 1.0000x reference)
import jax
import jax.numpy as jnp
from jax.experimental import pallas as pl


def kernel(x, conv_w, conv_b, bn_gamma, bn_beta):
    raise NotImplementedError("write your pallas kernel here")


# speedup vs baseline: 1.0310x; 1.0310x over previous
import functools

import jax
import jax.numpy as jnp
from jax import lax
from jax.experimental import pallas as pl
from jax.experimental.pallas import tpu as pltpu


def _stats_kernel(x_ref, g_ref, rs_ref):
    @pl.when(pl.program_id(1) == 0)
    def _init():
        g_ref[...] = jnp.zeros_like(g_ref)
        rs_ref[...] = jnp.zeros_like(rs_ref)

    xb = x_ref[0]
    g_ref[0] += lax.dot_general(xb, xb, (((1,), (1,)), ((), ())),
                                preferred_element_type=jnp.float32)
    rs_ref[0] += jnp.sum(xb, axis=1, keepdims=True)


def _apply_kernel(x_ref, tap_ref, g_ref, rs_ref, gam_ref, bet_ref, b_ref,
                  o_ref, w2_ref, sh_ref, *, nr, m_count, eps, slope):
    k_dim = w2_ref.shape[0]

    @pl.when(pl.program_id(1) == 0)
    def _prep():
        row = lax.broadcasted_iota(jnp.int32, (k_dim, k_dim), 0)
        col = lax.broadcasted_iota(jnp.int32, (k_dim, k_dim), 1)
        diff = (col - row) & (nr - 1)
        w = jnp.where(diff == nr - 1, tap_ref[0], 0.0)
        w = w + jnp.where(diff == 0, tap_ref[1], 0.0)
        w = w + jnp.where(diff == 1, tap_ref[2], 0.0)

        g = g_ref[0] + g_ref[1]
        rs = rs_ref[0] + rs_ref[1]
        t = jnp.dot(w, g, preferred_element_type=jnp.float32)
        s2raw = jnp.sum(t * w, axis=1, keepdims=True)
        sraw = jnp.dot(w, rs, preferred_element_type=jnp.float32)

        pool = jnp.where((row // nr) == (col // nr), 1.0, 0.0)
        s_p = jnp.dot(pool, sraw, preferred_element_type=jnp.float32)
        s2_p = jnp.dot(pool, s2raw, preferred_element_type=jnp.float32)

        b = b_ref[...]
        s = s_p + m_count * b
        s2 = s2_p + 2.0 * b * s_p + m_count * b * b
        mean = s / m_count
        var = jnp.maximum(s2 / m_count - mean * mean, 0.0)
        scale = gam_ref[...] * lax.rsqrt(var + eps)
        sh_ref[...] = scale * (b - mean) + bet_ref[...]
        w2_ref[...] = w * scale

    y = jnp.dot(w2_ref[...], x_ref[0], preferred_element_type=jnp.float32)
    y = y + sh_ref[...]
    o_ref[0] = jnp.maximum(y, slope * y).astype(o_ref.dtype)


def kernel(x, conv_w, conv_b, bn_gamma, bn_beta, *, eps=1e-5, slope=0.1):
    B, C, Nr, Np = x.shape
    K = C * Nr
    M = B * Np * Nr
    f32 = jnp.float32
    assert Nr & (Nr - 1) == 0, "ring dim assumed power of two"

    xf = x.reshape(B, K, Np)
    tap = jnp.broadcast_to(
        conv_w.astype(f32).transpose(2, 0, 1)[:, :, None, :, None],
        (3, C, Nr, C, Nr)).reshape(3, K, K)
    b_col = jnp.repeat(conv_b.astype(f32), Nr).reshape(K, 1)
    gam_col = jnp.repeat(bn_gamma.astype(f32), Nr).reshape(K, 1)
    bet_col = jnp.repeat(bn_beta.astype(f32), Nr).reshape(K, 1)

    NC = 2
    SB = B // NC
    x_spec = pl.BlockSpec((1, K, Np), lambda c, j: (c * SB + j, 0, 0))
    params = pltpu.CompilerParams(
        dimension_semantics=("parallel", "arbitrary"),
        vmem_limit_bytes=48 << 20)

    gp, rsp = pl.pallas_call(
        _stats_kernel,
        grid=(NC, SB),
        in_specs=[x_spec],
        out_specs=[pl.BlockSpec((1, K, K), lambda c, j: (c, 0, 0)),
                   pl.BlockSpec((1, K, 1), lambda c, j: (c, 0, 0))],
        out_shape=[jax.ShapeDtypeStruct((NC, K, K), f32),
                   jax.ShapeDtypeStruct((NC, K, 1), f32)],
        compiler_params=params,
    )(xf)

    const2 = lambda c, j: (0, 0)
    const3 = lambda c, j: (0, 0, 0)
    out_flat = pl.pallas_call(
        functools.partial(_apply_kernel, nr=Nr, m_count=float(M),
                          eps=eps, slope=slope),
        grid=(NC, SB),
        in_specs=[x_spec,
                  pl.BlockSpec((3, K, K), const3),
                  pl.BlockSpec((NC, K, K), const3),
                  pl.BlockSpec((NC, K, 1), const3),
                  pl.BlockSpec((K, 1), const2),
                  pl.BlockSpec((K, 1), const2),
                  pl.BlockSpec((K, 1), const2)],
        out_specs=x_spec,
        out_shape=jax.ShapeDtypeStruct((B, K, Np), x.dtype),
        scratch_shapes=[pltpu.VMEM((K, K), f32), pltpu.VMEM((K, 1), f32)],
        compiler_params=params,
    )(xf, tap, gp, rsp, gam_col, bet_col, b_col)
    return out_flat.reshape(B, C, Nr, Np)


# speedup vs baseline: 2.2714x; 2.2030x over previous
import functools

import jax
import jax.numpy as jnp
from jax import lax
from jax.experimental import pallas as pl
from jax.experimental.pallas import tpu as pltpu


def _fused_kernel(x_ref, tap_ref, gam_ref, bet_ref, b_ref, o_ref,
                  xs_ref, g_ref, rs_ref, w2_ref, sh_ref,
                  *, nr, bb, m_count, eps, slope):
    ph = pl.program_id(0)
    j = pl.program_id(1)
    k_dim = g_ref.shape[0]

    @pl.when((ph == 0) & (j == 0))
    def _init():
        g_ref[...] = jnp.zeros_like(g_ref)
        rs_ref[...] = jnp.zeros_like(rs_ref)

    @pl.when(ph == 0)
    def _stats():
        for i in range(bb):
            xb = x_ref[i]
            g_ref[...] += lax.dot_general(xb, xb, (((1,), (1,)), ((), ())),
                                          preferred_element_type=jnp.float32)
            rs_ref[...] += jnp.sum(xb, axis=1, keepdims=True)
            xs_ref[bb * j + i] = xb.astype(jnp.bfloat16)

    @pl.when((ph == 1) & (j == 0))
    def _prep():
        row = lax.broadcasted_iota(jnp.int32, (k_dim, k_dim), 0)
        col = lax.broadcasted_iota(jnp.int32, (k_dim, k_dim), 1)
        diff = (col - row) & (nr - 1)
        w = jnp.where(diff == nr - 1, tap_ref[0], 0.0)
        w = w + jnp.where(diff == 0, tap_ref[1], 0.0)
        w = w + jnp.where(diff == 1, tap_ref[2], 0.0)

        t = jnp.dot(w, g_ref[...], preferred_element_type=jnp.float32)
        s2raw = jnp.sum(t * w, axis=1, keepdims=True)
        sraw = jnp.dot(w, rs_ref[...], preferred_element_type=jnp.float32)

        pool = jnp.where((row // nr) == (col // nr), 1.0, 0.0)
        s_p = jnp.dot(pool, sraw, preferred_element_type=jnp.float32)
        s2_p = jnp.dot(pool, s2raw, preferred_element_type=jnp.float32)

        b = b_ref[...]
        s = s_p + m_count * b
        s2 = s2_p + 2.0 * b * s_p + m_count * b * b
        mean = s / m_count
        var = jnp.maximum(s2 / m_count - mean * mean, 0.0)
        scale = gam_ref[...] * lax.rsqrt(var + eps)
        sh_ref[...] = scale * (b - mean) + bet_ref[...]
        w2_ref[...] = w * scale

    @pl.when(ph == 1)
    def _apply():
        for i in range(bb):
            xb16 = xs_ref[bb * j + i]
            y = jnp.dot(w2_ref[...], xb16, preferred_element_type=jnp.float32)
            y = y + sh_ref[...]
            o_ref[i] = jnp.maximum(y, slope * y).astype(o_ref.dtype)


def kernel(x, conv_w, conv_b, bn_gamma, bn_beta, *, eps=1e-5, slope=0.1):
    B, C, Nr, Np = x.shape
    K = C * Nr
    M = B * Np * Nr
    f32 = jnp.float32
    assert Nr & (Nr - 1) == 0, "ring dim assumed power of two"

    xf = x.reshape(B, K, Np)
    tap = jnp.broadcast_to(
        conv_w.astype(f32).transpose(2, 0, 1)[:, :, None, :, None],
        (3, C, Nr, C, Nr)).reshape(3, K, K)
    b_col = jnp.repeat(conv_b.astype(f32), Nr).reshape(K, 1)
    gam_col = jnp.repeat(bn_gamma.astype(f32), Nr).reshape(K, 1)
    bet_col = jnp.repeat(bn_beta.astype(f32), Nr).reshape(K, 1)

    BB = 4
    J = B // BB
    x_spec = pl.BlockSpec((BB, K, Np),
                          lambda ph, j: (jnp.where(ph == 0, j, J - 1), 0, 0))
    o_spec = pl.BlockSpec((BB, K, Np),
                          lambda ph, j: (jnp.where(ph == 1, j, 0), 0, 0))
    const2 = lambda ph, j: (0, 0)
    const3 = lambda ph, j: (0, 0, 0)

    out_flat = pl.pallas_call(
        functools.partial(_fused_kernel, nr=Nr, bb=BB, m_count=float(M),
                          eps=eps, slope=slope),
        grid=(2, J),
        in_specs=[x_spec,
                  pl.BlockSpec((3, K, K), const3),
                  pl.BlockSpec((K, 1), const2),
                  pl.BlockSpec((K, 1), const2),
                  pl.BlockSpec((K, 1), const2)],
        out_specs=o_spec,
        out_shape=jax.ShapeDtypeStruct((B, K, Np), x.dtype),
        scratch_shapes=[pltpu.VMEM((B, K, Np), jnp.bfloat16),
                        pltpu.VMEM((K, K), f32),
                        pltpu.VMEM((K, 1), f32),
                        pltpu.VMEM((K, K), f32),
                        pltpu.VMEM((K, 1), f32)],
        compiler_params=pltpu.CompilerParams(
            dimension_semantics=("arbitrary", "arbitrary"),
            vmem_limit_bytes=56 << 20),
    )(xf, tap, gam_col, bet_col, b_col)
    return out_flat.reshape(B, C, Nr, Np)
